# baseline (speedup 1.0000x reference)
import jax
import jax.numpy as jnp
from jax.experimental import pallas as pl
from jax.experimental.pallas import tpu as pltpu


_CHUNK = 512


def _sum_kernel(x_ref, o_ref):
    @pl.when(pl.program_id(1) == 0)
    def _init():
        o_ref[...] = jnp.zeros_like(o_ref)

    o_ref[...] += jnp.sum(x_ref[...], axis=1, keepdims=True)


def kernel(inputs):
    b, l, d = inputs.shape
    grid = (b, l // _CHUNK)
    out = pl.pallas_call(
        _sum_kernel,
        grid=grid,
        in_specs=[pl.BlockSpec((1, _CHUNK, d), lambda i, j: (i, j, 0))],
        out_specs=pl.BlockSpec((1, 1, d), lambda i, j: (i, 0, 0)),
        out_shape=jax.ShapeDtypeStruct((b, 1, d), inputs.dtype),
        compiler_params=pltpu.CompilerParams(
            dimension_semantics=("parallel", "arbitrary"),
        ),
    )(inputs)
    return out.reshape(b, d)


# speedup vs baseline: 1.4692x; 1.4692x over previous
import jax
import jax.numpy as jnp
from jax.experimental import pallas as pl
from jax.experimental.pallas import tpu as pltpu


_CHUNK = 2048


def _sum_kernel(x_ref, o_ref):
    @pl.when(pl.program_id(1) == 0)
    def _init():
        o_ref[...] = jnp.zeros_like(o_ref)

    o_ref[...] += jnp.sum(x_ref[...], axis=1, keepdims=True)


def kernel(inputs):
    b, l, d = inputs.shape
    grid = (b, l // _CHUNK)
    out = pl.pallas_call(
        _sum_kernel,
        grid=grid,
        in_specs=[pl.BlockSpec((1, _CHUNK, d), lambda i, j: (i, j, 0))],
        out_specs=pl.BlockSpec((1, 1, d), lambda i, j: (i, 0, 0)),
        out_shape=jax.ShapeDtypeStruct((b, 1, d), inputs.dtype),
        compiler_params=pltpu.CompilerParams(
            dimension_semantics=("parallel", "arbitrary"),
        ),
    )(inputs)
    return out.reshape(b, d)
